# Initial kernel scaffold; baseline (speedup 1.0000x reference)
import functools

import jax
import jax.numpy as jnp
from jax import lax
from jax.experimental import pallas as pl
from jax.experimental.pallas import tpu as pltpu
from jax.experimental.pallas import tpu_sc as plsc

NC = 2
NS = 16
L = 16
BS = 128


def _sc_partials(ids, values, weights, n_rays):
    n, d = values.shape
    n_tiles = NC * NS
    per_tile = n // n_tiles
    nblk = per_tile // BS
    rows_per_sub = n_rays // NS

    mesh = plsc.VectorSubcoreMesh(core_axis_name="c", subcore_axis_name="s")

    @functools.partial(
        pl.kernel,
        out_type=jax.ShapeDtypeStruct((NC, n_rays, d), jnp.float32),
        mesh=mesh,
        scratch_types=[
            pltpu.VMEM((BS,), jnp.int32),
            pltpu.VMEM((BS,), jnp.float32),
            pltpu.VMEM((BS, d), jnp.float32),
            pltpu.VMEM_SHARED((n_rays, d), jnp.float32),
        ],
    )
    def k(ids_hbm, val_hbm, w_hbm, out_hbm, ids_v, w_v, vals_v, acc):
        c = lax.axis_index("c")
        s = lax.axis_index("s")
        tile = c * NS + s
        base = tile * per_tile

        def _zrow(i, carry):
            for j in range(d // L):
                vals_v[i, pl.ds(j * L, L)] = jnp.zeros((L,), jnp.float32)
            return carry
        lax.fori_loop(0, BS, _zrow, 0)
        for kk in range(rows_per_sub // BS):
            pltpu.sync_copy(vals_v, acc.at[pl.ds(s * rows_per_sub + kk * BS, BS)])
        plsc.subcore_barrier()

        def _block(b, carry):
            off = base + b * BS
            pltpu.sync_copy(ids_hbm.at[pl.ds(off, BS)], ids_v)
            pltpu.sync_copy(w_hbm.at[pl.ds(off, BS)], w_v)
            pltpu.sync_copy(val_hbm.at[pl.ds(off, BS)], vals_v)
            for g in range(BS // L):
                wv = w_v[pl.ds(g * L, L)]
                for j in range(L):
                    i = g * L + j
                    wsplat = jnp.take(wv, jnp.full((L,), j, jnp.int32),
                                      mode="promise_in_bounds")
                    for dd in range(d // L):
                        sl = pl.ds(dd * L, L)
                        vals_v[i, sl] = vals_v[i, sl] * wsplat
            pltpu.sync_copy(vals_v, acc.at[ids_v], add=True)
            return carry
        lax.fori_loop(0, nblk, _block, 0)

        plsc.subcore_barrier()
        r0 = s * rows_per_sub
        pltpu.sync_copy(acc.at[pl.ds(r0, rows_per_sub)],
                        out_hbm.at[c, pl.ds(r0, rows_per_sub)])

    return k(ids, values, weights)


def _combine(partials):
    _, n_rays, d = partials.shape
    blk = 1024

    def body(p_ref, o_ref):
        o_ref[...] = p_ref[0] + p_ref[1]

    return pl.pallas_call(
        body,
        grid=(n_rays // blk,),
        in_specs=[pl.BlockSpec((2, blk, d), lambda i: (0, i, 0))],
        out_specs=pl.BlockSpec((blk, d), lambda i: (i, 0)),
        out_shape=jax.ShapeDtypeStruct((n_rays, d), jnp.float32),
    )(partials)


def kernel(ray_samples_packed, value_samples, weights_samples):
    n_rays = 8192
    ids = ray_samples_packed.astype(jnp.int32)
    w = weights_samples.reshape(-1).astype(jnp.float32)
    partials = _sc_partials(ids, value_samples, w, n_rays)
    return _combine(partials)


# speedup vs baseline: 1.4934x; 1.4934x over previous
import functools

import jax
import jax.numpy as jnp
from jax import lax
from jax.experimental import pallas as pl
from jax.experimental.pallas import tpu as pltpu
from jax.experimental.pallas import tpu_sc as plsc

NC = 2
NS = 16
L = 16
C = 64
N_RAYS = 8192
RPT = N_RAYS // (NC * NS)


def _splat(vec, j):
    return lax.gather(
        vec, jnp.full((L, 1), j, jnp.int32),
        lax.GatherDimensionNumbers(offset_dims=(), collapsed_slice_dims=(0,),
                                   start_index_map=(0,)),
        (1,), mode=lax.GatherScatterMode.PROMISE_IN_BOUNDS)


def _extract_lane(vec, lane, nbits=19):
    m = lax.iota(jnp.int32, L) == lane
    v = jnp.int32(0)
    for bit in range(nbits):
        has = jnp.any(jnp.logical_and(((vec >> bit) & 1) == 1, m))
        v = v + jnp.where(has, jnp.int32(1 << bit), jnp.int32(0))
    return v


def _sc_segment_sum(bounds, ids, values, weights):
    n_pad, d = values.shape

    mesh = plsc.VectorSubcoreMesh(core_axis_name="c", subcore_axis_name="s",
                                  num_cores=NC, num_subcores=NS)

    @functools.partial(
        pl.kernel,
        out_type=jax.ShapeDtypeStruct((N_RAYS, d), jnp.float32),
        mesh=mesh,
        compiler_params=pltpu.CompilerParams(needs_layout_passes=False),
        scratch_types=[
            pltpu.VMEM((L,), jnp.int32),
            pltpu.VMEM((C,), jnp.int32),
            pltpu.VMEM((C,), jnp.float32),
            pltpu.VMEM((C, d), jnp.float32),
            pltpu.VMEM((RPT, d), jnp.float32),
        ],
    )
    def k(b_hbm, ids_hbm, w_hbm, val_hbm, out_hbm, b_v, ids_v, w_v, vals_v,
          acc):
        c = lax.axis_index("c")
        s = lax.axis_index("s")
        t = c * NS + s
        ray_lo = t * RPT

        def _zero(i, carry):
            for j in range(d // L):
                acc[i, pl.ds(j * L, L)] = jnp.zeros((L,), jnp.float32)
            return carry
        lax.fori_loop(0, RPT, _zero, 0)

        pltpu.sync_copy(b_hbm.at[t], b_v)
        bv = b_v[...]
        s_lo = _extract_lane(bv, 0)
        s_hi = _extract_lane(bv, 1)
        off0 = (s_lo // 8) * 8
        nblk = (s_hi - off0 + C - 1) // C

        io = lax.iota(jnp.int32, L)

        def _block(b, carry):
            off = off0 + b * C
            pltpu.sync_copy(ids_hbm.at[pl.ds(off, C)], ids_v)
            pltpu.sync_copy(w_hbm.at[pl.ds(off, C)], w_v)
            pltpu.sync_copy(val_hbm.at[pl.ds(off, C)], vals_v)
            for g in range(C // L):
                gidx = io + (off + g * L)
                valid = jnp.logical_and(gidx >= s_lo, gidx < s_hi)
                wv = jnp.where(valid, w_v[pl.ds(g * L, L)],
                               jnp.zeros((L,), jnp.float32))
                idv = ids_v[pl.ds(g * L, L)] - ray_lo
                idv = jnp.minimum(jnp.maximum(idv, 0), RPT - 1)
                for j in range(L):
                    i = g * L + j
                    idsp = _splat(idv, j)
                    wsp = _splat(wv, j)
                    for cc in range(d // L):
                        x = wsp * vals_v[i, pl.ds(cc * L, L)]
                        plsc.addupdate_scatter(acc, [idsp, io + cc * L], x)
            return carry
        lax.fori_loop(0, nblk, _block, 0)

        pltpu.sync_copy(acc, out_hbm.at[pl.ds(ray_lo, RPT)])

    return k(bounds, ids, weights, values)


def kernel(ray_samples_packed, value_samples, weights_samples):
    ids = ray_samples_packed.astype(jnp.int32)
    w = weights_samples.reshape(-1).astype(jnp.float32)
    vals = value_samples.astype(jnp.float32)

    edges = jnp.searchsorted(ids, jnp.arange(33, dtype=jnp.int32) * RPT
                             ).astype(jnp.int32)
    brow = jnp.zeros((NC * NS, L), jnp.int32)
    brow = brow.at[:, 0].set(edges[:-1])
    brow = brow.at[:, 1].set(edges[1:])

    pad = 2 * C
    ids_p = jnp.concatenate([ids, jnp.full((pad,), N_RAYS - 1, jnp.int32)])
    w_p = jnp.concatenate([w, jnp.zeros((pad,), jnp.float32)])
    vals_p = jnp.concatenate(
        [vals, jnp.zeros((pad, vals.shape[1]), jnp.float32)])

    return _sc_segment_sum(brow, ids_p, vals_p, w_p)


# speedup vs baseline: 1.5048x; 1.0076x over previous
import functools

import jax
import jax.numpy as jnp
from jax import lax
from jax.experimental import pallas as pl
from jax.experimental.pallas import tpu as pltpu
from jax.experimental.pallas import tpu_sc as plsc

NC = 2
NS = 16
L = 16
C = 64
N_RAYS = 8192
RPT = N_RAYS // (NC * NS)


def _splat(vec, j):
    return lax.gather(
        vec, jnp.full((L, 1), j, jnp.int32),
        lax.GatherDimensionNumbers(offset_dims=(), collapsed_slice_dims=(0,),
                                   start_index_map=(0,)),
        (1,), mode=lax.GatherScatterMode.PROMISE_IN_BOUNDS)


def _extract_lane(vec, lane, nbits=19):
    m = lax.iota(jnp.int32, L) == lane
    v = jnp.int32(0)
    for bit in range(nbits):
        has = jnp.any(jnp.logical_and(((vec >> bit) & 1) == 1, m))
        v = v + jnp.where(has, jnp.int32(1 << bit), jnp.int32(0))
    return v


def _sc_segment_sum(bounds, ids, values, weights):
    n_pad, d = values.shape

    mesh = plsc.VectorSubcoreMesh(core_axis_name="c", subcore_axis_name="s",
                                  num_cores=NC, num_subcores=NS)

    @functools.partial(
        pl.kernel,
        out_type=jax.ShapeDtypeStruct((N_RAYS, d), jnp.float32),
        mesh=mesh,
        compiler_params=pltpu.CompilerParams(needs_layout_passes=False),
        scratch_types=[
            pltpu.VMEM((L,), jnp.int32),
            pltpu.VMEM((C,), jnp.int32),
            pltpu.VMEM((C,), jnp.float32),
            pltpu.VMEM((C, d), jnp.float32),
            pltpu.VMEM((RPT, d), jnp.float32),
        ],
    )
    def k(b_hbm, ids_hbm, w_hbm, val_hbm, out_hbm, b_v, ids_v, w_v, vals_v,
          acc):
        c = lax.axis_index("c")
        s = lax.axis_index("s")
        t = c * NS + s
        ray_lo = t * RPT

        def _zero(i, carry):
            for j in range(d // L):
                acc[i, pl.ds(j * L, L)] = jnp.zeros((L,), jnp.float32)
            return carry
        lax.fori_loop(0, RPT, _zero, 0)

        pltpu.sync_copy(b_hbm.at[t], b_v)
        bv = b_v[...]
        s_lo = _extract_lane(bv, 0)
        s_hi = _extract_lane(bv, 1)
        off0 = (s_lo // 8) * 8
        nblk = (s_hi - off0 + C - 1) // C

        io = lax.iota(jnp.int32, L)

        def _block(b, carry):
            off = off0 + b * C
            pltpu.sync_copy(ids_hbm.at[pl.ds(off, C)], ids_v)
            pltpu.sync_copy(w_hbm.at[pl.ds(off, C)], w_v)
            pltpu.sync_copy(val_hbm.at[pl.ds(off, C)], vals_v)
            for g in range(C // L):
                gidx = io + (off + g * L)
                valid = jnp.logical_and(gidx >= s_lo, gidx < s_hi)
                wv = jnp.where(valid, w_v[pl.ds(g * L, L)],
                               jnp.zeros((L,), jnp.float32))
                idv = ids_v[pl.ds(g * L, L)] - ray_lo
                idv = jnp.minimum(jnp.maximum(idv, 0), RPT - 1)
                id0 = _splat(idv, 0)
                uniform = jnp.logical_not(jnp.any(idv != id0))

                def _fast(_):
                    wsp = [_splat(wv, j) for j in range(L)]
                    for cc in range(d // L):
                        sl = pl.ds(cc * L, L)
                        x = wsp[0] * vals_v[g * L, sl]
                        for j in range(1, L):
                            x = x + wsp[j] * vals_v[g * L + j, sl]
                        plsc.addupdate_scatter(acc, [id0, io + cc * L], x)
                    return 0

                def _slow(_):
                    for j in range(L):
                        i = g * L + j
                        idsp = _splat(idv, j)
                        wsp = _splat(wv, j)
                        for cc in range(d // L):
                            x = wsp * vals_v[i, pl.ds(cc * L, L)]
                            plsc.addupdate_scatter(acc, [idsp, io + cc * L], x)
                    return 0

                lax.cond(uniform, _fast, _slow, 0)
            return carry
        lax.fori_loop(0, nblk, _block, 0)

        pltpu.sync_copy(acc, out_hbm.at[pl.ds(ray_lo, RPT)])

    return k(bounds, ids, weights, values)


def kernel(ray_samples_packed, value_samples, weights_samples):
    ids = ray_samples_packed.astype(jnp.int32)
    w = weights_samples.reshape(-1).astype(jnp.float32)
    vals = value_samples.astype(jnp.float32)

    edges = jnp.searchsorted(ids, jnp.arange(33, dtype=jnp.int32) * RPT
                             ).astype(jnp.int32)
    brow = jnp.zeros((NC * NS, L), jnp.int32)
    brow = brow.at[:, 0].set(edges[:-1])
    brow = brow.at[:, 1].set(edges[1:])

    pad = 2 * C
    ids_p = jnp.concatenate([ids, jnp.full((pad,), N_RAYS - 1, jnp.int32)])
    w_p = jnp.concatenate([w, jnp.zeros((pad,), jnp.float32)])
    vals_p = jnp.concatenate(
        [vals, jnp.zeros((pad, vals.shape[1]), jnp.float32)])

    return _sc_segment_sum(brow, ids_p, vals_p, w_p)
